# Initial kernel scaffold; baseline (speedup 1.0000x reference)
import jax
import jax.numpy as jnp
from jax.experimental import pallas as pl


def kernel(x, w1, b1, w_bbox, b_bbox, w_cls, b_cls):
    raise NotImplementedError("write your pallas kernel here")


# speedup vs baseline: 7.8510x; 7.8510x over previous
import numpy as np
import jax
import jax.numpy as jnp
from jax.experimental import pallas as pl
from jax.experimental.pallas import tpu as pltpu

_TH = 8
_H, _W = 48, 80
_PRE = 2000
_PREP = 2048
_POST = 300
_THR = 0.7
_NEG = -1e30


def _anchor_consts():
    base_size = 16
    xc = yc = 0.5 * (base_size - 1)
    size = float(base_size * base_size)
    rows = []
    for r in (0.5, 1.0, 2.0):
        ws = np.round(np.sqrt(size / r))
        hs = np.round(ws * r)
        for s in (8, 16, 32):
            w, h = ws * s, hs * s
            rows.append([xc - 0.5 * (w - 1), yc - 0.5 * (h - 1),
                         xc + 0.5 * (w - 1), yc + 0.5 * (h - 1)])
    anc = np.array(rows, np.float64)
    aw = anc[:, 2] - anc[:, 0] + 1.0
    ah = anc[:, 3] - anc[:, 1] + 1.0
    acx = anc[:, 0] + 0.5 * aw
    acy = anc[:, 1] + 0.5 * ah
    c = np.zeros((8, 128), np.float32)
    c[0, :9] = acx
    c[1, :9] = acy
    c[2, :9] = aw
    c[3, :9] = ah
    return c


def _conv1_body(xee_r, xeen_r, xeo_r, xeon_r, xoe_r, xoo_r, w_r, b_r, o_r):
    ee = xee_r[0]
    een = xeen_r[0]
    eo = xeo_r[0]
    eon = xeon_r[0]
    oe = xoe_r[0]
    oo = xoo_r[0]
    ee1 = jnp.concatenate([ee[1:], een[:1]], axis=0)
    eo1 = jnp.concatenate([eo[1:], eon[:1]], axis=0)
    taps = (ee[:, 0:_W], eo[:, 0:_W], ee[:, 1:_W + 1],
            oe[:, 0:_W], oo[:, 0:_W], oe[:, 1:_W + 1],
            ee1[:, 0:_W], eo1[:, 0:_W], ee1[:, 1:_W + 1])
    acc = None
    for k, t in enumerate(taps):
        m = t.reshape(_TH * _W, 512)
        p = jax.lax.dot_general(m, w_r[k], (((1,), (0,)), ((), ())),
                                preferred_element_type=jnp.float32)
        acc = p if acc is None else acc + p
    o_r[0] = acc + b_r[...]


def _head_body(f_r, w_r, b_r, c_r, x1_r, y1_r, x2_r, y2_r, s_r, pad_r):
    pad_r[...] = jnp.zeros_like(pad_r)
    pad_r[1:_H + 1, 1:_W + 1, :] = f_r[0].reshape(_H, _W, 512)
    acc = None
    for ky in range(3):
        for kx in range(3):
            t = pad_r[ky:ky + _H, kx:kx + _W, :].reshape(_H * _W, 512)
            for h in (0, 1):
                p = jax.lax.dot_general(
                    t[:, h * 256:(h + 1) * 256],
                    w_r[ky * 3 + kx, h * 256:(h + 1) * 256, :],
                    (((1,), (0,)), ((), ())),
                    preferred_element_type=jnp.float32)
                acc = p if acc is None else acc + p
    head = acc + b_r[...]

    sub = jax.lax.broadcasted_iota(jnp.int32, (_H * _W, 1), 0)
    jj = (sub % _W).astype(jnp.float32)
    ii = (sub // _W).astype(jnp.float32)
    acx = c_r[0:1, 0:9]
    acy = c_r[1:2, 0:9]
    aw = c_r[2:3, 0:9]
    ah = c_r[3:4, 0:9]
    cx = jj * 32.0 + acx
    cy = ii * 32.0 + acy
    dx = head[:, 0:9]
    dy = head[:, 16:25]
    dw = head[:, 32:41]
    dh = head[:, 48:57]
    sc = head[:, 64:73]
    pcx = dx * aw + cx
    pcy = dy * ah + cy
    pw = jnp.exp(dw) * aw
    ph = jnp.exp(dh) * ah
    x1 = pcx - 0.5 * pw
    y1 = pcy - 0.5 * ph
    x2 = pcx + 0.5 * pw
    y2 = pcy + 0.5 * ph
    limx = float(_W * 32 - 1)
    limy = float(_H * 32 - 1)
    x1 = jnp.minimum(jnp.maximum(x1, 0.0), limx)
    y1 = jnp.minimum(jnp.maximum(y1, 0.0), limy)
    x2 = jnp.minimum(jnp.maximum(x2, 0.0), limx)
    y2 = jnp.minimum(jnp.maximum(y2, 0.0), limy)
    bw = x2 - x1 + 1.0
    bh = y2 - y1 + 1.0
    sc = jnp.where((bw >= 16.0) & (bh >= 16.0), sc, _NEG)
    x1_r[0] = x1
    y1_r[0] = y1
    x2_r[0] = x2
    y2_r[0] = y2
    s_r[0] = sc


def _nms_body(x1_r, y1_r, x2_r, y2_r, s_r, sel_r, num_r, val_r):
    b = s_r.shape[0]
    val_r[...] = jnp.where(s_r[...] > _NEG * 0.5, 1.0, 0.0)
    num_r[...] = jnp.zeros_like(num_r)
    lane = jax.lax.broadcasted_iota(jnp.int32, (b, _PREP), 1)
    prio0 = (_PREP - lane).astype(jnp.float32)

    def step(t, carry):
        valid = val_r[...]
        pr = jnp.where(valid > 0.0, prio0, 0.0)
        m = jnp.max(pr, axis=1, keepdims=True)
        anyv = m > 0.0
        pickv = jnp.min(jnp.where(pr == m, lane, _PREP), axis=1, keepdims=True)
        oh = jnp.where(lane == pickv, 1.0, 0.0)
        x1 = x1_r[...]
        y1 = y1_r[...]
        x2 = x2_r[...]
        y2 = y2_r[...]
        px1 = jnp.sum(x1 * oh, axis=1, keepdims=True)
        py1 = jnp.sum(y1 * oh, axis=1, keepdims=True)
        px2 = jnp.sum(x2 * oh, axis=1, keepdims=True)
        py2 = jnp.sum(y2 * oh, axis=1, keepdims=True)
        area = (x2 - x1 + 1.0) * (y2 - y1 + 1.0)
        parea = (px2 - px1 + 1.0) * (py2 - py1 + 1.0)
        iw = jnp.maximum(0.0, jnp.minimum(x2, px2) - jnp.maximum(x1, px1) + 1.0)
        ih = jnp.maximum(0.0, jnp.minimum(y2, py2) - jnp.maximum(y1, py1) + 1.0)
        inter = iw * ih
        iou = inter / (area + parea - inter)
        val_r[...] = jnp.where(valid > 0.0,
                               jnp.where(iou <= _THR, 1.0, 0.0), 0.0)
        fl = jnp.where(anyv, 1.0, 0.0)
        row = jnp.concatenate([px1, py1, px2, py2], axis=1) * fl
        sel_r[t] = row
        num_r[...] = num_r[...] + fl
        return carry

    jax.lax.fori_loop(0, _POST, step, 0)


def kernel(x, w1, b1, w_bbox, b_bbox, w_cls, b_cls):
    B = x.shape[0]
    f32 = jnp.float32

    xt = jnp.transpose(x, (0, 2, 3, 1))
    xp = jnp.pad(xt, ((0, 0), (1, 1), (1, 1), (0, 0)))
    xee = xp[:, 0::2, 0::2, :]
    xeo = xp[:, 0::2, 1::2, :]
    xoe = xp[:, 1::2, 0::2, :][:, :_H]
    xoo = xp[:, 1::2, 1::2, :][:, :_H]
    xee = jnp.pad(xee, ((0, 0), (0, 7), (0, 0), (0, 0)))
    xeo = jnp.pad(xeo, ((0, 0), (0, 7), (0, 0), (0, 0)))
    wt = jnp.transpose(w1, (2, 3, 1, 0)).reshape(9, 512, 512)

    n_rt = _H // _TH
    xspec = lambda f: pl.BlockSpec((1, _TH, _W + 1, 512), f)
    feat = pl.pallas_call(
        _conv1_body,
        grid=(B, n_rt),
        in_specs=[
            xspec(lambda b, r: (b, r, 0, 0)),
            xspec(lambda b, r: (b, r + 1, 0, 0)),
            xspec(lambda b, r: (b, r, 0, 0)),
            xspec(lambda b, r: (b, r + 1, 0, 0)),
            xspec(lambda b, r: (b, r, 0, 0)),
            xspec(lambda b, r: (b, r, 0, 0)),
            pl.BlockSpec((9, 512, 512), lambda b, r: (0, 0, 0)),
            pl.BlockSpec((1, 512), lambda b, r: (0, 0)),
        ],
        out_specs=pl.BlockSpec((1, _TH * _W, 512), lambda b, r: (b, r, 0)),
        out_shape=jax.ShapeDtypeStruct((B, _H * _W, 512), f32),
        compiler_params=pltpu.CompilerParams(
            dimension_semantics=("parallel", "arbitrary"),
            vmem_limit_bytes=56 * 1024 * 1024),
        name="rpn_conv1",
    )(xee, xee, xeo, xeo, xoe, xoo, wt, b1.reshape(1, 512))

    wb = w_bbox[:, :, 0, 0]
    dest_d = np.concatenate([16 * g + np.arange(9) for g in range(4)])
    src_d = np.concatenate([np.arange(9) * 4 + g for g in range(4)])
    wh = jnp.zeros((9, 512, 128), f32)
    wh = wh.at[4, :, dest_d].set(wb[src_d])
    wc = jnp.transpose(w_cls[9:18], (2, 3, 1, 0)).reshape(9, 512, 9)
    wh = wh.at[:, :, 64:73].set(wc)
    bh = jnp.zeros((128,), f32).at[dest_d].set(b_bbox[src_d])
    bh = bh.at[64:73].set(b_cls[9:18])
    anc_c = jnp.asarray(_anchor_consts())

    ospec = jax.ShapeDtypeStruct((B, _H * _W, 9), f32)
    x1a, y1a, x2a, y2a, sca = pl.pallas_call(
        _head_body,
        grid=(B,),
        in_specs=[
            pl.BlockSpec((1, _H * _W, 512), lambda b: (b, 0, 0)),
            pl.BlockSpec((9, 512, 128), lambda b: (0, 0, 0)),
            pl.BlockSpec((1, 128), lambda b: (0, 0)),
            pl.BlockSpec((8, 128), lambda b: (0, 0)),
        ],
        out_specs=[pl.BlockSpec((1, _H * _W, 9), lambda b: (b, 0, 0))] * 5,
        out_shape=[ospec] * 5,
        scratch_shapes=[pltpu.VMEM((_H + 2, _W + 2, 512), f32)],
        compiler_params=pltpu.CompilerParams(
            dimension_semantics=("parallel",),
            vmem_limit_bytes=56 * 1024 * 1024),
        name="rpn_heads",
    )(feat, wh, bh.reshape(1, 128), anc_c)

    n = _H * _W * 9
    top_s, idx = jax.lax.top_k(sca.reshape(B, n), _PRE)
    gath = lambda a: jnp.take_along_axis(a.reshape(B, n), idx, axis=1)
    padc = lambda a: jnp.pad(a, ((0, 0), (0, _PREP - _PRE)))
    gx1, gy1, gx2, gy2 = (padc(gath(a)) for a in (x1a, y1a, x2a, y2a))
    gs = jnp.pad(top_s, ((0, 0), (0, _PREP - _PRE)), constant_values=_NEG)

    sel, numf = pl.pallas_call(
        _nms_body,
        out_shape=[jax.ShapeDtypeStruct((_POST, B, 4), f32),
                   jax.ShapeDtypeStruct((B, 1), f32)],
        scratch_shapes=[pltpu.VMEM((B, _PREP), f32)],
        name="rpn_nms",
    )(gx1, gy1, gx2, gy2, gs)

    sel = sel.transpose(1, 0, 2)
    num = numf[:, 0].astype(jnp.int32)
    bidx = jnp.broadcast_to(jnp.arange(B, dtype=sel.dtype)[:, None, None],
                            (B, _POST, 1))
    rois = jnp.concatenate([bidx, sel], axis=-1)
    return rois, num
